# baseline (speedup 1.0000x reference)
import functools

import jax
import jax.numpy as jnp
from jax.experimental import pallas as pl
from jax.experimental.pallas import tpu as pltpu


def _gru_block_kernel(len_sref, x_ref, wih_ref, whh_ref, bxi_ref, bhn_ref,
                      len_ref, out_ref, h_ref, xi_ref, *, tblk, b, h):
    i = pl.program_id(0)

    @pl.when(i == 0)
    def _init():
        h_ref[...] = jnp.zeros_like(h_ref)

    t0 = i * tblk
    maxlen = len_sref[0]
    nrem = jnp.clip(maxlen - t0, 0, tblk)
    nsteps = jnp.minimum((nrem + 3) & ~3, tblk)

    @pl.when(nrem < tblk)
    def _zero():
        out_ref[...] = jnp.zeros_like(out_ref)

    @pl.when(nrem > 0)
    def _work():
        xt = jnp.transpose(x_ref[...], (2, 0, 1))
        xblk = xt.reshape(tblk * b, x_ref.shape[1])
        xi = jnp.dot(xblk.astype(jnp.bfloat16), wih_ref[...],
                     preferred_element_type=jnp.float32)
        xi_ref[...] = (xi + bxi_ref[...]).reshape(tblk, b, 3 * h)

        whh = whh_ref[...]
        bhn = bhn_ref[...]
        lens = len_ref[...]

        def step(t, hcur):
            xi_t = xi_ref[t]
            gh = jnp.dot(hcur.astype(jnp.bfloat16), whh,
                         preferred_element_type=jnp.float32)
            tr = jnp.tanh(xi_t[:, :h] + gh[:, :h])
            tz = jnp.tanh(xi_t[:, h:2 * h] + gh[:, h:2 * h])
            hn2 = 0.5 * gh[:, 2 * h:] + bhn
            n = jnp.tanh((xi_t[:, 2 * h:] + hn2) + tr * hn2)
            zn = 0.5 - 0.5 * tz
            zh = (0.5 + 0.5 * tz) * hcur
            hnew = n * zn + zh
            mask = (lens > (t0 + t)).astype(hnew.dtype)
            out_ref[t] = hnew * mask
            return hnew

        def step4(j, hcur):
            return step(4 * j + 3, step(4 * j + 2,
                        step(4 * j + 1, step(4 * j, hcur))))

        h_ref[...] = jax.lax.fori_loop(0, nsteps // 4, step4, h_ref[...])


def kernel(x, lengths, W_ih, W_hh, b_ih, b_hh):
    B, C, T = x.shape
    H = W_hh.shape[1]
    TBLK = 512
    assert T % TBLK == 0

    scale = jnp.concatenate([jnp.full((2 * H,), 0.5, jnp.float32),
                             jnp.ones((H,), jnp.float32)])
    wih_t = (W_ih.T * scale[None, :]).astype(jnp.bfloat16)
    whh_t = (W_hh.T * scale[None, :]).astype(jnp.bfloat16)
    bxi = ((b_ih + jnp.concatenate([b_hh[:2 * H],
                                    jnp.zeros((H,), b_hh.dtype)])) * scale
           ).reshape(1, 3 * H)
    bhn = (0.5 * b_hh[2 * H:]).reshape(1, H)
    lens_i32 = lengths.astype(jnp.int32)
    lens2d = jnp.broadcast_to(lens_i32[:, None], (B, H))

    grid_spec = pltpu.PrefetchScalarGridSpec(
        num_scalar_prefetch=1,
        grid=(T // TBLK,),
        in_specs=[
            pl.BlockSpec((B, C, TBLK), lambda i, sref: (0, 0, i)),
            pl.BlockSpec((C, 3 * H), lambda i, sref: (0, 0)),
            pl.BlockSpec((H, 3 * H), lambda i, sref: (0, 0)),
            pl.BlockSpec((1, 3 * H), lambda i, sref: (0, 0)),
            pl.BlockSpec((1, H), lambda i, sref: (0, 0)),
            pl.BlockSpec((B, H), lambda i, sref: (0, 0)),
        ],
        out_specs=pl.BlockSpec((TBLK, B, H), lambda i, sref: (i, 0, 0)),
        scratch_shapes=[
            pltpu.VMEM((B, H), jnp.float32),
            pltpu.VMEM((TBLK, B, 3 * H), jnp.float32),
        ],
    )
    out_tbh = pl.pallas_call(
        functools.partial(_gru_block_kernel, tblk=TBLK, b=B, h=H),
        grid_spec=grid_spec,
        out_shape=jax.ShapeDtypeStruct((T, B, H), x.dtype),
        compiler_params=pltpu.CompilerParams(
            dimension_semantics=("arbitrary",),
        ),
    )(lens_i32, x, wih_t, whh_t, bxi, bhn, lens2d)

    out = _transpose2d(out_tbh.reshape(T, B * H)).reshape(B, H, T)
    return (out, lengths)


def _tr_kernel(in_ref, out_ref):
    out_ref[...] = in_ref[...].T


def _transpose2d(a):
    n, m = a.shape
    TB = 256
    return pl.pallas_call(
        _tr_kernel,
        grid=(n // TB, m // TB),
        in_specs=[pl.BlockSpec((TB, TB), lambda i, j: (i, j))],
        out_specs=pl.BlockSpec((TB, TB), lambda i, j: (j, i)),
        out_shape=jax.ShapeDtypeStruct((m, n), a.dtype),
    )(a)


# speedup vs baseline: 1.1448x; 1.1448x over previous
import functools

import jax
import jax.numpy as jnp
from jax.experimental import pallas as pl
from jax.experimental.pallas import tpu as pltpu


def _gru_block_kernel(len_sref, x_ref, wih_ref, whh_ref, bxi_ref, bhn_ref,
                      len_ref, out_ref, h_ref, xi_ref, *, tblk, b, h):
    i = pl.program_id(0)

    @pl.when(i == 0)
    def _init():
        h_ref[...] = jnp.zeros_like(h_ref)

    t0 = i * tblk
    maxlen = len_sref[0]
    nrem = jnp.clip(maxlen - t0, 0, tblk)
    nsteps = jnp.minimum((nrem + 7) & ~7, tblk)

    @pl.when(nrem < tblk)
    def _zero():
        out_ref[...] = jnp.zeros_like(out_ref)

    @pl.when(nrem > 0)
    def _work():
        xt = jnp.transpose(x_ref[...], (2, 0, 1))
        xblk = xt.reshape(tblk * b, x_ref.shape[1])
        xi = jnp.dot(xblk.astype(jnp.bfloat16), wih_ref[...],
                     preferred_element_type=jnp.float32)
        xi_ref[...] = (xi + bxi_ref[...]).reshape(tblk, b, 3 * h)

        whh = whh_ref[...]
        bhn = bhn_ref[...]
        lens = len_ref[...]

        def step(t, hcur):
            xi_t = xi_ref[t]
            gh = jnp.dot(hcur.astype(jnp.bfloat16), whh,
                         preferred_element_type=jnp.float32)
            tr = jnp.tanh(xi_t[:, :h] + gh[:, :h])
            tz = jnp.tanh(xi_t[:, h:2 * h] + gh[:, h:2 * h])
            hn2 = 0.5 * gh[:, 2 * h:] + bhn
            n = jnp.tanh((xi_t[:, 2 * h:] + hn2) + tr * hn2)
            zn = 0.5 - 0.5 * tz
            zh = (0.5 + 0.5 * tz) * hcur
            hnew = n * zn + zh
            mask = (lens > (t0 + t)).astype(hnew.dtype)
            out_ref[t] = hnew * mask
            return hnew

        def step8(j, hcur):
            for k in range(8):
                hcur = step(8 * j + k, hcur)
            return hcur

        h_ref[...] = jax.lax.fori_loop(0, nsteps // 8, step8, h_ref[...])


def kernel(x, lengths, W_ih, W_hh, b_ih, b_hh):
    B, C, T = x.shape
    H = W_hh.shape[1]
    TBLK = 512
    assert T % TBLK == 0

    scale = jnp.concatenate([jnp.full((2 * H,), 0.5, jnp.float32),
                             jnp.ones((H,), jnp.float32)])
    wih_t = (W_ih.T * scale[None, :]).astype(jnp.bfloat16)
    whh_t = (W_hh.T * scale[None, :]).astype(jnp.bfloat16)
    bxi = ((b_ih + jnp.concatenate([b_hh[:2 * H],
                                    jnp.zeros((H,), b_hh.dtype)])) * scale
           ).reshape(1, 3 * H)
    bhn = (0.5 * b_hh[2 * H:]).reshape(1, H)
    lens_i32 = lengths.astype(jnp.int32)
    lens2d = jnp.broadcast_to(lens_i32[:, None], (B, H))

    grid_spec = pltpu.PrefetchScalarGridSpec(
        num_scalar_prefetch=1,
        grid=(T // TBLK,),
        in_specs=[
            pl.BlockSpec((B, C, TBLK), lambda i, sref: (0, 0, i)),
            pl.BlockSpec((C, 3 * H), lambda i, sref: (0, 0)),
            pl.BlockSpec((H, 3 * H), lambda i, sref: (0, 0)),
            pl.BlockSpec((1, 3 * H), lambda i, sref: (0, 0)),
            pl.BlockSpec((1, H), lambda i, sref: (0, 0)),
            pl.BlockSpec((B, H), lambda i, sref: (0, 0)),
        ],
        out_specs=pl.BlockSpec((TBLK, B, H), lambda i, sref: (i, 0, 0)),
        scratch_shapes=[
            pltpu.VMEM((B, H), jnp.float32),
            pltpu.VMEM((TBLK, B, 3 * H), jnp.float32),
        ],
    )
    out_tbh = pl.pallas_call(
        functools.partial(_gru_block_kernel, tblk=TBLK, b=B, h=H),
        grid_spec=grid_spec,
        out_shape=jax.ShapeDtypeStruct((T, B, H), x.dtype),
        compiler_params=pltpu.CompilerParams(
            dimension_semantics=("arbitrary",),
        ),
    )(lens_i32, x, wih_t, whh_t, bxi, bhn, lens2d)

    out = jnp.transpose(out_tbh, (1, 2, 0))
    return (out, lengths)
